# baseline (speedup 1.0000x reference)
import jax
import jax.numpy as jnp
from jax.experimental import pallas as pl


def _trim_passthrough_kernel(x_ref, v_ref, m_ref, xo_ref, vo_ref, mo_ref):
    xo_ref[...] = x_ref[...]
    vo_ref[...] = v_ref[...]
    mo_ref[...] = (m_ref[...] != 0.0).astype(jnp.float32)


def kernel(x, v, mask):
    B = x.shape[0]
    out = pl.pallas_call(
        _trim_passthrough_kernel,
        grid=(B,),
        in_specs=[
            pl.BlockSpec((1,) + x.shape[1:], lambda i: (i, 0, 0)),
            pl.BlockSpec((1,) + v.shape[1:], lambda i: (i, 0, 0)),
            pl.BlockSpec((1,) + mask.shape[1:], lambda i: (i, 0, 0)),
        ],
        out_specs=[
            pl.BlockSpec((1,) + x.shape[1:], lambda i: (i, 0, 0)),
            pl.BlockSpec((1,) + v.shape[1:], lambda i: (i, 0, 0)),
            pl.BlockSpec((1,) + mask.shape[1:], lambda i: (i, 0, 0)),
        ],
        out_shape=[
            jax.ShapeDtypeStruct(x.shape, x.dtype),
            jax.ShapeDtypeStruct(v.shape, v.dtype),
            jax.ShapeDtypeStruct(mask.shape, jnp.float32),
        ],
    )(x, v, mask)
    return (out[0], out[1], out[2].astype(bool))


# speedup vs baseline: 1.0009x; 1.0009x over previous
import jax
import jax.numpy as jnp
from jax.experimental import pallas as pl


def _trim_passthrough_kernel(x_ref, v_ref, m_ref, xo_ref, vo_ref, mo_ref):
    xo_ref[...] = x_ref[...]
    vo_ref[...] = v_ref[...]
    mo_ref[...] = m_ref[...] != 0.0


def kernel(x, v, mask):
    B = x.shape[0]
    out = pl.pallas_call(
        _trim_passthrough_kernel,
        grid=(B,),
        in_specs=[
            pl.BlockSpec((1,) + x.shape[1:], lambda i: (i, 0, 0)),
            pl.BlockSpec((1,) + v.shape[1:], lambda i: (i, 0, 0)),
            pl.BlockSpec((1,) + mask.shape[1:], lambda i: (i, 0, 0)),
        ],
        out_specs=[
            pl.BlockSpec((1,) + x.shape[1:], lambda i: (i, 0, 0)),
            pl.BlockSpec((1,) + v.shape[1:], lambda i: (i, 0, 0)),
            pl.BlockSpec((1,) + mask.shape[1:], lambda i: (i, 0, 0)),
        ],
        out_shape=[
            jax.ShapeDtypeStruct(x.shape, x.dtype),
            jax.ShapeDtypeStruct(v.shape, v.dtype),
            jax.ShapeDtypeStruct(mask.shape, jnp.bool_),
        ],
    )(x, v, mask)
    return (out[0], out[1], out[2])
